# Initial kernel scaffold; baseline (speedup 1.0000x reference)
import jax
import jax.numpy as jnp
from jax.experimental import pallas as pl


def kernel(X, G):
    raise NotImplementedError("write your pallas kernel here")


# speedup vs baseline: 2.3000x; 2.3000x over previous
import functools

import jax
import jax.numpy as jnp
from jax import lax
from jax.experimental import pallas as pl
from jax.experimental.pallas import tpu as pltpu
from jax.experimental.pallas import tpu_sc as plsc

N, K, D = 10000, 32, 128
B = N * K
NC, NS = 2, 16
NW = NC * NS
B_PER_W = B // NW
CHUNK = 80
NCHUNK = B_PER_W // CHUNK


def _gather_sc(x, idx):
    mesh = plsc.VectorSubcoreMesh(core_axis_name="c", subcore_axis_name="s")

    @functools.partial(
        pl.kernel,
        mesh=mesh,
        out_type=jax.ShapeDtypeStruct((B, D), jnp.float32),
        scratch_types=[
            pltpu.VMEM((B_PER_W,), jnp.int32),
            pltpu.VMEM((CHUNK, D), jnp.float32),
            pltpu.SemaphoreType.DMA,
        ],
    )
    def k(x_hbm, idx_hbm, out_hbm, idx_v, buf, sem):
        wid = lax.axis_index("s") * NC + lax.axis_index("c")
        base = wid * B_PER_W
        pltpu.sync_copy(idx_hbm.at[pl.ds(base, B_PER_W)], idx_v)

        def body(i, carry):
            off = pl.multiple_of(i * CHUNK, 8)
            idx_c = idx_v.at[pl.ds(off, CHUNK)]
            pltpu.async_copy(x_hbm.at[idx_c], buf, sem).wait()
            pltpu.sync_copy(buf, out_hbm.at[pl.ds(base + off, CHUNK)])
            return carry

        lax.fori_loop(0, NCHUNK, body, 0)

    return k(x, idx)


def kernel(X, G):
    idx = G.reshape(-1).astype(jnp.int32)
    out = _gather_sc(X, idx)
    return out.reshape(N, K * D)


# speedup vs baseline: 3.0317x; 1.3181x over previous
import functools

import jax
import jax.numpy as jnp
from jax import lax
from jax.experimental import pallas as pl
from jax.experimental.pallas import tpu as pltpu
from jax.experimental.pallas import tpu_sc as plsc

N, K, D = 10000, 32, 128
B = N * K
NC, NS = 2, 16
NW = NC * NS
B_PER_W = B // NW
CHUNK = 80
NCHUNK = B_PER_W // CHUNK
NBUF = 5


def _gather_sc(x, idx):
    mesh = plsc.VectorSubcoreMesh(core_axis_name="c", subcore_axis_name="s")

    @functools.partial(
        pl.kernel,
        mesh=mesh,
        out_type=jax.ShapeDtypeStruct((B, D), jnp.float32),
        scratch_types=[
            pltpu.VMEM((B_PER_W,), jnp.int32),
        ]
        + [pltpu.VMEM((CHUNK, D), jnp.float32) for _ in range(NBUF)]
        + [pltpu.SemaphoreType.DMA for _ in range(NBUF)],
    )
    def k(x_hbm, idx_hbm, out_hbm, idx_v, *bufs_sems):
        bufs = bufs_sems[:NBUF]
        gsems = bufs_sems[NBUF:]
        wid = lax.axis_index("s") * NC + lax.axis_index("c")
        base = wid * B_PER_W
        pltpu.sync_copy(idx_hbm.at[pl.ds(base, B_PER_W)], idx_v)

        def g_copy(i, b):
            off = pl.multiple_of(i * CHUNK, 8)
            return pltpu.make_async_copy(
                x_hbm.at[idx_v.at[pl.ds(off, CHUNK)]], bufs[b], gsems[b])

        for b in range(NBUF):
            g_copy(b, b).start()

        def body(g, carry):
            for b in range(NBUF):
                i = g * NBUF + b
                g_copy(i, b).wait()
                off = pl.multiple_of(base + i * CHUNK, 8)
                pltpu.sync_copy(bufs[b], out_hbm.at[pl.ds(off, CHUNK)])
                j = i + NBUF

                @pl.when(j < NCHUNK)
                def _():
                    g_copy(j, b).start()

            return carry

        lax.fori_loop(0, NCHUNK // NBUF, body, 0)

    return k(x, idx)


def kernel(X, G):
    idx = G.reshape(-1).astype(jnp.int32)
    out = _gather_sc(X, idx)
    return out.reshape(N, K * D)
